# baseline (speedup 1.0000x reference)
import jax
import jax.numpy as jnp
from jax import lax
from jax.experimental import pallas as pl
from jax.experimental.pallas import tpu as pltpu
from jax.experimental.pallas import tpu_sc as plsc

_SIZE = 1000000
_CHUNK = 16
_NUM_HASHES = 2
_RANGE = _SIZE - _CHUNK
_ROWS = _SIZE // _CHUNK

_B, _F = 16384, 26
_N = _B * _F
_NC, _NS, _L = 2, 16, 16
_NW = _NC * _NS
_W = _N // _NW
_K = 512
_NBLK = _W // _K
_NIDX = 2 * _K
_RA = _SIZE // 32
_RB = (_SIZE - 32) // 32

_DNUMS = lax.GatherDimensionNumbers(
    offset_dims=(), collapsed_slice_dims=(0,), start_index_map=(0,))


def _dyng(v, idx):
    return lax.gather(v, idx[:, None], _DNUMS, (1,),
                      mode=lax.GatherScatterMode.PROMISE_IN_BOUNDS)


def _body(x_hbm, tab_hbm, co_hbm, out_hbm, x_v, co_v,
          idx_a, idx_b, stage_a, stage_b, s0_a, s0_b, s1_a, s1_b,
          out_a, out_b, sem_ga, sem_gb, sem_oa, sem_ob):
    wid = lax.axis_index("s") * _NC + lax.axis_index("c")
    base_in = wid * _W
    pltpu.sync_copy(co_hbm, co_v)
    pltpu.sync_copy(x_hbm.at[pl.ds(base_in, _W)], x_v)
    c0 = co_v[0]
    c1 = co_v[1]
    iota = lax.iota(jnp.int32, _L)
    rng = jnp.uint32(_RANGE)

    bufs = ((idx_a, stage_a, s0_a, s1_a, out_a, sem_ga, sem_oa),
            (idx_b, stage_b, s0_b, s1_b, out_b, sem_gb, sem_ob))

    def hash_block(b, parity):
        idx_v, _, s0_v, s1_v, _, _, _ = bufs[parity]

        def hash_step(i2, _):
            sl = pl.ds(i2 * _L, _L)
            xu = x_v[pl.ds(b * _K + i2 * _L, _L)].astype(jnp.uint32)
            hv0 = ((xu * c0) % rng).astype(jnp.int32)
            hv1 = ((xu * c1) % rng).astype(jnp.int32)
            s50 = hv0 & 31
            s51 = hv1 & 31
            wrap0 = jnp.where(s50 > 16, 1, 0)
            wrap1 = jnp.where(s51 > 16, 1, 0)
            idx_v[sl] = (hv0 >> 5) + wrap0 * _RA
            idx_v[pl.ds(_K + i2 * _L, _L)] = (hv1 >> 5) + wrap1 * _RA
            s0_v[sl] = s50 - wrap0 * 16
            s1_v[sl] = s51 - wrap1 * 16
            return _

        lax.fori_loop(0, _K // _L, hash_step, None)

    def fire_gather(b, parity):
        idx_v, stage_v, _, _, _, sem_g, _ = bufs[parity]
        for j in range(_NIDX // 128):
            pltpu.async_copy(tab_hbm.at[idx_v.at[pl.ds(j * 128, 128)]],
                             stage_v.at[pl.ds(j * 128, 128)], sem_g)

    def extract_block(parity):
        _, stage_v, s0_v, s1_v, out_v, _, _ = bufs[parity]

        @plsc.parallel_loop(0, _K // _L)
        def _extract(g):
            s0vec = s0_v[pl.ds(g * _L, _L)]
            s1vec = s1_v[pl.ds(g * _L, _L)]
            for j in range(_L):
                ii = g * _L + j
                for h, svec in ((0, s0vec), (1, s1vec)):
                    sv = jnp.full((_L,), svec[j], jnp.int32)
                    v_lo = stage_v[h * _K + ii, 0]
                    v_hi = stage_v[h * _K + ii, 1]
                    comb = jnp.where(iota >= sv, v_lo, v_hi)
                    vals = _dyng(comb, (sv + iota) & 15)
                    out_v[pl.ds(32 * ii + 16 * h, _L)] = vals

    def fire_out(b, parity):
        _, _, _, _, out_v, _, sem_o = bufs[parity]
        pltpu.async_copy(
            out_v, out_hbm.at[pl.ds((base_in + b * _K) * 32, _K * 32)], sem_o)

    def wait_gather(parity):
        idx_v, stage_v, _, _, _, sem_g, _ = bufs[parity]
        for j in range(_NIDX // 128):
            pltpu.make_async_copy(
                tab_hbm.at[idx_v.at[pl.ds(j * 128, 128)]],
                stage_v.at[pl.ds(j * 128, 128)], sem_g).wait()

    def wait_out(parity):
        _, _, _, _, out_v, _, sem_o = bufs[parity]
        pltpu.make_async_copy(
            out_v, out_hbm.at[pl.ds(0, _K * 32)], sem_o).wait()

    npair = _NBLK // 2
    hash_block(0, 0)
    fire_gather(0, 0)

    def pair(i, _):
        b0 = 2 * i
        hash_block(b0 + 1, 1)
        fire_gather(b0 + 1, 1)
        wait_gather(0)

        @pl.when(i > 0)
        def _():
            wait_out(0)

        extract_block(0)
        fire_out(b0, 0)

        @pl.when(i < npair - 1)
        def _():
            hash_block(b0 + 2, 0)
            fire_gather(b0 + 2, 0)

        wait_gather(1)

        @pl.when(i > 0)
        def _():
            wait_out(1)

        extract_block(1)
        fire_out(b0 + 1, 1)
        return _

    lax.fori_loop(0, npair, pair, None)
    wait_out(0)
    wait_out(1)


def kernel(input_tensor, table, hash_coeffs):
    x = input_tensor.reshape(_N)
    tab = jnp.concatenate(
        [table.reshape(_RA, 32), table[16:16 + _RB * 32].reshape(_RB, 32)],
        axis=0).reshape(_RA + _RB, 2, _CHUNK)
    co = jnp.broadcast_to(hash_coeffs.reshape(_NUM_HASHES, 1),
                          (_NUM_HASHES, _L)).astype(jnp.uint32)
    mesh = plsc.VectorSubcoreMesh(core_axis_name="c", subcore_axis_name="s")
    dbuf = lambda shape, dtype: [pltpu.VMEM(shape, dtype)] * 2
    run = pl.kernel(
        _body,
        out_type=jax.ShapeDtypeStruct((_N * 32,), jnp.float32),
        mesh=mesh,
        compiler_params=pltpu.CompilerParams(use_tc_tiling_on_sc=False),
        scratch_types=[
            pltpu.VMEM((_W,), jnp.int32),
            pltpu.VMEM((_NUM_HASHES, _L), jnp.uint32),
            *dbuf((_NIDX,), jnp.int32),
            *dbuf((_NIDX, 2, _CHUNK), jnp.float32),
            *dbuf((_K,), jnp.int32),
            *dbuf((_K,), jnp.int32),
            *dbuf((_K * 32,), jnp.float32),
            pltpu.SemaphoreType.DMA,
            pltpu.SemaphoreType.DMA,
            pltpu.SemaphoreType.DMA,
            pltpu.SemaphoreType.DMA,
        ],
    )
    out = run(x, tab, co)
    return out.reshape(_B, _F, _NUM_HASHES * _CHUNK)


# speedup vs baseline: 1.7843x; 1.7843x over previous
import jax
import jax.numpy as jnp
from jax import lax
from jax.experimental import pallas as pl
from jax.experimental.pallas import tpu as pltpu
from jax.experimental.pallas import tpu_sc as plsc

_SIZE = 1000000
_CHUNK = 16
_NUM_HASHES = 2
_RANGE = _SIZE - _CHUNK
_ROWS = _SIZE // _CHUNK

_B, _F = 16384, 26
_N = _B * _F
_NC, _NS, _L = 2, 16, 16
_NW = _NC * _NS
_W = _N // _NW
_K = 512
_NBLK = _W // _K
_NIDX = 2 * _K
_RA = _SIZE // 32
_RB = (_SIZE - 32) // 32

_DNUMS = lax.GatherDimensionNumbers(
    offset_dims=(), collapsed_slice_dims=(0,), start_index_map=(0,))


def _dyng(v, idx):
    return lax.gather(v, idx[:, None], _DNUMS, (1,),
                      mode=lax.GatherScatterMode.PROMISE_IN_BOUNDS)


def _body(x_hbm, tab_hbm, co_hbm, out_hbm, x_v, co_v,
          idx_a, idx_b, stage_a, stage_b, s0_a, s0_b, s1_a, s1_b,
          out_a, out_b, sem_ga, sem_gb, sem_oa, sem_ob):
    wid = lax.axis_index("s") * _NC + lax.axis_index("c")
    base_in = wid * _W
    pltpu.sync_copy(co_hbm, co_v)
    pltpu.sync_copy(x_hbm.at[pl.ds(base_in, _W)], x_v)
    c0 = co_v[0]
    c1 = co_v[1]
    iota = lax.iota(jnp.int32, _L)
    rng = jnp.uint32(_RANGE)

    bufs = ((idx_a, stage_a, s0_a, s1_a, out_a, sem_ga, sem_oa),
            (idx_b, stage_b, s0_b, s1_b, out_b, sem_gb, sem_ob))

    def hash_block(b, parity):
        idx_v, _, s0_v, s1_v, _, _, _ = bufs[parity]

        def hash_step(i2, _):
            sl = pl.ds(i2 * _L, _L)
            xu = x_v[pl.ds(b * _K + i2 * _L, _L)].astype(jnp.uint32)
            hv0 = ((xu * c0) % rng).astype(jnp.int32)
            hv1 = ((xu * c1) % rng).astype(jnp.int32)
            s50 = hv0 & 31
            s51 = hv1 & 31
            wrap0 = jnp.where(s50 > 16, 1, 0)
            wrap1 = jnp.where(s51 > 16, 1, 0)
            idx_v[sl] = (hv0 >> 5) + wrap0 * _RA
            idx_v[pl.ds(_K + i2 * _L, _L)] = (hv1 >> 5) + wrap1 * _RA
            s0_v[sl] = s50 - wrap0 * 16
            s1_v[sl] = s51 - wrap1 * 16
            return _

        lax.fori_loop(0, _K // _L, hash_step, None)

    def fire_gather(b, parity):
        idx_v, stage_v, _, _, _, sem_g, _ = bufs[parity]
        for j in range(_NIDX // 128):
            pltpu.async_copy(tab_hbm.at[idx_v.at[pl.ds(j * 128, 128)]],
                             stage_v.at[pl.ds(j * 128, 128)], sem_g)

    def extract_block(parity):
        _, stage_v, s0_v, s1_v, out_v, _, _ = bufs[parity]

        @plsc.parallel_loop(0, _K // _L)
        def _extract(g):
            s0vec = s0_v[pl.ds(g * _L, _L)]
            s1vec = s1_v[pl.ds(g * _L, _L)]
            for j in range(_L):
                ii = g * _L + j
                for h, svec in ((0, s0vec), (1, s1vec)):
                    sv = jnp.full((_L,), svec[j], jnp.int32)
                    v_lo = stage_v[h * _K + ii, pl.ds(0, _L)]
                    v_hi = stage_v[h * _K + ii, pl.ds(_L, _L)]
                    comb = jnp.where(iota >= sv, v_lo, v_hi)
                    vals = _dyng(comb, (sv + iota) & 15)
                    out_v[pl.ds(32 * ii + 16 * h, _L)] = vals

    def fire_out(b, parity):
        _, _, _, _, out_v, _, sem_o = bufs[parity]
        pltpu.async_copy(
            out_v, out_hbm.at[pl.ds((base_in + b * _K) * 32, _K * 32)], sem_o)

    def wait_gather(parity):
        idx_v, stage_v, _, _, _, sem_g, _ = bufs[parity]
        for j in range(_NIDX // 128):
            pltpu.make_async_copy(
                tab_hbm.at[idx_v.at[pl.ds(j * 128, 128)]],
                stage_v.at[pl.ds(j * 128, 128)], sem_g).wait()

    def wait_out(parity):
        _, _, _, _, out_v, _, sem_o = bufs[parity]
        pltpu.make_async_copy(
            out_v, out_hbm.at[pl.ds(0, _K * 32)], sem_o).wait()

    npair = _NBLK // 2
    hash_block(0, 0)
    fire_gather(0, 0)

    def pair(i, _):
        b0 = 2 * i
        hash_block(b0 + 1, 1)
        fire_gather(b0 + 1, 1)
        wait_gather(0)

        @pl.when(i > 0)
        def _():
            wait_out(0)

        extract_block(0)
        fire_out(b0, 0)

        @pl.when(i < npair - 1)
        def _():
            hash_block(b0 + 2, 0)
            fire_gather(b0 + 2, 0)

        wait_gather(1)

        @pl.when(i > 0)
        def _():
            wait_out(1)

        extract_block(1)
        fire_out(b0 + 1, 1)
        return _

    lax.fori_loop(0, npair, pair, None)
    wait_out(0)
    wait_out(1)


def kernel(input_tensor, table, hash_coeffs):
    x = input_tensor.reshape(_N)
    tab = jnp.concatenate(
        [table.reshape(_RA, 32), table[16:16 + _RB * 32].reshape(_RB, 32)],
        axis=0)
    co = jnp.broadcast_to(hash_coeffs.reshape(_NUM_HASHES, 1),
                          (_NUM_HASHES, _L)).astype(jnp.uint32)
    mesh = plsc.VectorSubcoreMesh(core_axis_name="c", subcore_axis_name="s")
    dbuf = lambda shape, dtype: [pltpu.VMEM(shape, dtype)] * 2
    run = pl.kernel(
        _body,
        out_type=jax.ShapeDtypeStruct((_N * 32,), jnp.float32),
        mesh=mesh,
        compiler_params=pltpu.CompilerParams(use_tc_tiling_on_sc=False),
        scratch_types=[
            pltpu.VMEM((_W,), jnp.int32),
            pltpu.VMEM((_NUM_HASHES, _L), jnp.uint32),
            *dbuf((_NIDX,), jnp.int32),
            *dbuf((_NIDX, 32), jnp.float32),
            *dbuf((_K,), jnp.int32),
            *dbuf((_K,), jnp.int32),
            *dbuf((_K * 32,), jnp.float32),
            pltpu.SemaphoreType.DMA,
            pltpu.SemaphoreType.DMA,
            pltpu.SemaphoreType.DMA,
            pltpu.SemaphoreType.DMA,
        ],
    )
    out = run(x, tab, co)
    return out.reshape(_B, _F, _NUM_HASHES * _CHUNK)


# speedup vs baseline: 2.0841x; 1.1680x over previous
import jax
import jax.numpy as jnp
from jax import lax
from jax.experimental import pallas as pl
from jax.experimental.pallas import tpu as pltpu
from jax.experimental.pallas import tpu_sc as plsc

_SIZE = 1000000
_CHUNK = 16
_NUM_HASHES = 2
_RANGE = _SIZE - _CHUNK
_ROWS = _SIZE // _CHUNK

_B, _F = 16384, 26
_N = _B * _F
_NC, _NS, _L = 2, 16, 16
_NW = _NC * _NS
_W = _N // _NW
_K = 512
_NBLK = _W // _K
_NIDX = 4 * _K

_DNUMS = lax.GatherDimensionNumbers(
    offset_dims=(), collapsed_slice_dims=(0,), start_index_map=(0,))


def _dyng(v, idx):
    return lax.gather(v, idx[:, None], _DNUMS, (1,),
                      mode=lax.GatherScatterMode.PROMISE_IN_BOUNDS)


def _body(x_hbm, tab_hbm, co_hbm, out_hbm, x_v, co_v,
          idx_a, idx_b, stage_a, stage_b, s0_a, s0_b, s1_a, s1_b,
          out_a, out_b, sem_ga, sem_gb, sem_oa, sem_ob):
    wid = lax.axis_index("s") * _NC + lax.axis_index("c")
    base_in = wid * _W
    pltpu.sync_copy(co_hbm, co_v)
    pltpu.sync_copy(x_hbm.at[pl.ds(base_in, _W)], x_v)
    c0 = co_v[0]
    c1 = co_v[1]
    iota = lax.iota(jnp.int32, _L)
    rng = jnp.uint32(_RANGE)

    bufs = ((idx_a, stage_a, s0_a, s1_a, out_a, sem_ga, sem_oa),
            (idx_b, stage_b, s0_b, s1_b, out_b, sem_gb, sem_ob))

    def hash_block(b, parity):
        idx_v, _, s0_v, s1_v, _, _, _ = bufs[parity]

        def hash_step(i2, _):
            sl = pl.ds(i2 * _L, _L)
            xu = x_v[pl.ds(b * _K + i2 * _L, _L)].astype(jnp.uint32)
            hv0 = ((xu * c0) % rng).astype(jnp.int32)
            hv1 = ((xu * c1) % rng).astype(jnp.int32)
            r0 = hv0 >> 4
            r1 = hv1 >> 4
            idx_v[sl] = r0
            idx_v[pl.ds(_K + i2 * _L, _L)] = r0 + 1
            idx_v[pl.ds(2 * _K + i2 * _L, _L)] = r1
            idx_v[pl.ds(3 * _K + i2 * _L, _L)] = r1 + 1
            s0_v[sl] = hv0 & 15
            s1_v[sl] = hv1 & 15
            return _

        lax.fori_loop(0, _K // _L, hash_step, None)

    def fire_gather(b, parity):
        idx_v, stage_v, _, _, _, sem_g, _ = bufs[parity]
        for j in range(_NIDX // 128):
            pltpu.async_copy(tab_hbm.at[idx_v.at[pl.ds(j * 128, 128)]],
                             stage_v.at[pl.ds(j * 128, 128)], sem_g)

    def extract_block(parity):
        _, stage_v, s0_v, s1_v, out_v, _, _ = bufs[parity]

        @plsc.parallel_loop(0, _K // _L)
        def _extract(g):
            s0vec = s0_v[pl.ds(g * _L, _L)]
            s1vec = s1_v[pl.ds(g * _L, _L)]
            for j in range(_L):
                ii = g * _L + j
                for h, svec in ((0, s0vec), (1, s1vec)):
                    sv = jnp.full((_L,), svec[j], jnp.int32)
                    v_lo = stage_v[2 * h * _K + ii]
                    v_hi = stage_v[(2 * h + 1) * _K + ii]
                    comb = jnp.where(iota >= sv, v_lo, v_hi)
                    vals = _dyng(comb, (sv + iota) & 15)
                    out_v[pl.ds(32 * ii + 16 * h, _L)] = vals

    def fire_out(b, parity):
        _, _, _, _, out_v, _, sem_o = bufs[parity]
        pltpu.async_copy(
            out_v, out_hbm.at[pl.ds((base_in + b * _K) * 32, _K * 32)], sem_o)

    def wait_gather(parity):
        idx_v, stage_v, _, _, _, sem_g, _ = bufs[parity]
        for j in range(_NIDX // 128):
            pltpu.make_async_copy(
                tab_hbm.at[idx_v.at[pl.ds(j * 128, 128)]],
                stage_v.at[pl.ds(j * 128, 128)], sem_g).wait()

    def wait_out(parity):
        _, _, _, _, out_v, _, sem_o = bufs[parity]
        pltpu.make_async_copy(
            out_v, out_hbm.at[pl.ds(0, _K * 32)], sem_o).wait()

    npair = _NBLK // 2
    hash_block(0, 0)
    fire_gather(0, 0)

    def pair(i, _):
        b0 = 2 * i
        hash_block(b0 + 1, 1)
        fire_gather(b0 + 1, 1)
        wait_gather(0)

        @pl.when(i > 0)
        def _():
            wait_out(0)

        extract_block(0)
        fire_out(b0, 0)

        @pl.when(i < npair - 1)
        def _():
            hash_block(b0 + 2, 0)
            fire_gather(b0 + 2, 0)

        wait_gather(1)

        @pl.when(i > 0)
        def _():
            wait_out(1)

        extract_block(1)
        fire_out(b0 + 1, 1)
        return _

    lax.fori_loop(0, npair, pair, None)
    wait_out(0)
    wait_out(1)


def kernel(input_tensor, table, hash_coeffs):
    x = input_tensor.reshape(_N)
    tab = table.reshape(_ROWS, _CHUNK)
    co = jnp.broadcast_to(hash_coeffs.reshape(_NUM_HASHES, 1),
                          (_NUM_HASHES, _L)).astype(jnp.uint32)
    mesh = plsc.VectorSubcoreMesh(core_axis_name="c", subcore_axis_name="s")
    dbuf = lambda shape, dtype: [pltpu.VMEM(shape, dtype)] * 2
    run = pl.kernel(
        _body,
        out_type=jax.ShapeDtypeStruct((_N * 32,), jnp.float32),
        mesh=mesh,
        compiler_params=pltpu.CompilerParams(use_tc_tiling_on_sc=False),
        scratch_types=[
            pltpu.VMEM((_W,), jnp.int32),
            pltpu.VMEM((_NUM_HASHES, _L), jnp.uint32),
            *dbuf((_NIDX,), jnp.int32),
            *dbuf((_NIDX, _CHUNK), jnp.float32),
            *dbuf((_K,), jnp.int32),
            *dbuf((_K,), jnp.int32),
            *dbuf((_K * 32,), jnp.float32),
            pltpu.SemaphoreType.DMA,
            pltpu.SemaphoreType.DMA,
            pltpu.SemaphoreType.DMA,
            pltpu.SemaphoreType.DMA,
        ],
    )
    out = run(x, tab, co)
    return out.reshape(_B, _F, _NUM_HASHES * _CHUNK)
